# baseline (speedup 1.0000x reference)
import functools

import jax
import jax.numpy as jnp
from jax import lax
from jax.experimental import pallas as pl
from jax.experimental.pallas import tpu as pltpu
from jax.experimental.pallas import tpu_sc as plsc

N = 10000
D_IN = 128
D_HID = 16
E = 320000

NW = 32
EPW = E // NW
CH = 128
K = 3
FULL = EPW // CH
G = FULL // K
TAIL = EPW - FULL * CH
NP = 10112
STRIPE = NP // 16
NP1 = 10240
STRIPE1 = NP1 // 16

_SC_MESH = plsc.VectorSubcoreMesh(core_axis_name="c", subcore_axis_name="s")
_SC_PARAMS = pltpu.CompilerParams(use_tc_tiling_on_sc=False)


def _zero_fill(ref, nrows):
    def body(i, _):
        ref[i] = jnp.zeros((D_HID,), jnp.float32)
        return 0

    lax.fori_loop(0, nrows, body, 0)


@functools.partial(
    pl.kernel,
    out_type=jax.ShapeDtypeStruct((2, NP1), jnp.float32),
    mesh=_SC_MESH,
    scratch_types=[
        [pltpu.VMEM((CH,), jnp.int32) for _ in range(2 * K)],
        pltpu.VMEM((CH,), jnp.float32),
        pltpu.VMEM((STRIPE1,), jnp.float32),
        pltpu.VMEM_SHARED((NP1,), jnp.float32),
        [pltpu.SemaphoreType.DMA for _ in range(2)],
        [pltpu.SemaphoreType.DMA for _ in range(K)],
    ],
    compiler_params=_SC_PARAMS,
)
def _sc_degree(dst_hbm, out_hbm, dstb, ones_v, wb_v, acc, si, ss):
    c = lax.axis_index("c")
    s = lax.axis_index("s")
    base = s * STRIPE1

    def zfill(i, _):
        wb_v[pl.ds(i * 16, 16)] = jnp.zeros((16,), jnp.float32)
        return 0

    lax.fori_loop(0, STRIPE1 // 16, zfill, 0)
    pltpu.sync_copy(wb_v, acc.at[pl.ds(base, STRIPE1)])

    def fill_ones(i, _):
        ones_v[pl.ds(i * 16, 16)] = jnp.ones((16,), jnp.float32)
        return 0

    lax.fori_loop(0, CH // 16, fill_ones, 0)
    plsc.subcore_barrier()

    ebase = (c * 16 + s) * EPW

    def idx_src(g, b):
        return dst_hbm.at[pl.ds(ebase + (g * K + b) * CH, CH)]

    for b in range(K):
        pltpu.async_copy(idx_src(0, b), dstb[b], si[0])

    def phase(g, g2, p, first, last):
        for b in range(K):
            pltpu.make_async_copy(idx_src(g, b), dstb[p * K + b], si[p]).wait()

        def wait_scatters():
            for b in range(K):
                pltpu.make_async_copy(ones_v, acc.at[dstb[p * K + b]],
                                      ss[b]).wait()

        if first:
            pl.when(g2 > 0)(wait_scatters)
        else:
            wait_scatters()

        for b in range(K):
            pltpu.async_copy(ones_v, acc.at[dstb[p * K + b]], ss[b], add=True)

        def prefetch():
            for b in range(K):
                pltpu.async_copy(idx_src(g + 1, b), dstb[(1 - p) * K + b],
                                 si[1 - p])

        if last:
            pl.when(g2 + 1 < G // 2)(prefetch)
        else:
            prefetch()

    def step(g2, _):
        phase(2 * g2, g2, 0, True, False)
        phase(2 * g2 + 1, g2, 1, False, True)
        return 0

    lax.fori_loop(0, G // 2, step, 0)
    for b in range(K):
        pltpu.make_async_copy(ones_v, acc.at[dstb[b]], ss[b]).wait()

    pltpu.sync_copy(dst_hbm.at[pl.ds(ebase + FULL * CH, TAIL)],
                    dstb[0].at[pl.ds(0, TAIL)])
    pltpu.sync_copy(ones_v.at[pl.ds(0, TAIL)],
                    acc.at[dstb[0].at[pl.ds(0, TAIL)]], add=True)

    plsc.subcore_barrier()
    pltpu.sync_copy(acc.at[pl.ds(base, STRIPE1)], wb_v)
    pltpu.sync_copy(wb_v, out_hbm.at[c].at[pl.ds(base, STRIPE1)])


@functools.partial(
    pl.kernel,
    out_type=jax.ShapeDtypeStruct((2, NP1), jnp.float32),
    mesh=_SC_MESH,
    scratch_types=[
        [pltpu.VMEM((CH,), jnp.int32) for _ in range(2 * K)],
        [pltpu.VMEM((CH,), jnp.int32) for _ in range(2 * K)],
        [pltpu.VMEM((CH,), jnp.float32) for _ in range(K)],
        pltpu.VMEM((TAIL,), jnp.int32),
        pltpu.VMEM((TAIL,), jnp.int32),
        pltpu.VMEM((TAIL,), jnp.float32),
        pltpu.VMEM((STRIPE1,), jnp.float32),
        pltpu.VMEM_SHARED((NP1,), jnp.float32),
        [pltpu.SemaphoreType.DMA for _ in range(2)],
        [pltpu.SemaphoreType.DMA for _ in range(K)],
        [pltpu.SemaphoreType.DMA for _ in range(K)],
    ],
    compiler_params=_SC_PARAMS,
)
def _sc_aggregate1(tab_hbm, src_hbm, dst_hbm, out_hbm, srcb, dstb, rows, srct,
                   dstt, rowst, wb_v, acc, si, sg, ss):
    c = lax.axis_index("c")
    s = lax.axis_index("s")
    base = s * STRIPE1

    def zfill(i, _):
        wb_v[pl.ds(i * 16, 16)] = jnp.zeros((16,), jnp.float32)
        return 0

    lax.fori_loop(0, STRIPE1 // 16, zfill, 0)
    pltpu.sync_copy(wb_v, acc.at[pl.ds(base, STRIPE1)])
    plsc.subcore_barrier()

    ebase = (c * 16 + s) * EPW

    def idx_at(row, g, b):
        ref = src_hbm if row == 0 else dst_hbm
        return ref.at[pl.ds(ebase + (g * K + b) * CH, CH)]

    for b in range(K):
        pltpu.async_copy(idx_at(0, 0, b), srcb[b], si[0])
        pltpu.async_copy(idx_at(1, 0, b), dstb[b], si[0])

    def phase(g, g2, p, first, last):
        for b in range(K):
            pltpu.make_async_copy(idx_at(0, g, b), srcb[p * K + b], si[p]).wait()
            pltpu.make_async_copy(idx_at(1, g, b), dstb[p * K + b], si[p]).wait()

        def wait_scatters():
            for b in range(K):
                pltpu.make_async_copy(rows[b], acc.at[dstb[p * K + b]],
                                      ss[b]).wait()

        if first:
            pl.when(g2 > 0)(wait_scatters)
        else:
            wait_scatters()

        for b in range(K):
            pltpu.async_copy(tab_hbm.at[srcb[p * K + b]], rows[b], sg[b])

        def prefetch():
            for b in range(K):
                pltpu.async_copy(idx_at(0, g + 1, b), srcb[(1 - p) * K + b],
                                 si[1 - p])
                pltpu.async_copy(idx_at(1, g + 1, b), dstb[(1 - p) * K + b],
                                 si[1 - p])

        if last:
            pl.when(g2 + 1 < G // 2)(prefetch)
        else:
            prefetch()

        for b in range(K):
            pltpu.make_async_copy(tab_hbm.at[srcb[p * K + b]], rows[b],
                                  sg[b]).wait()
            pltpu.async_copy(rows[b], acc.at[dstb[p * K + b]], ss[b], add=True)

    def step(g2, _):
        phase(2 * g2, g2, 0, True, False)
        phase(2 * g2 + 1, g2, 1, False, True)
        return 0

    lax.fori_loop(0, G // 2, step, 0)
    for b in range(K):
        pltpu.make_async_copy(rows[b], acc.at[dstb[b]], ss[b]).wait()

    pltpu.sync_copy(src_hbm.at[pl.ds(ebase + FULL * CH, TAIL)], srct)
    pltpu.sync_copy(dst_hbm.at[pl.ds(ebase + FULL * CH, TAIL)], dstt)
    pltpu.sync_copy(tab_hbm.at[srct], rowst)
    pltpu.sync_copy(rowst, acc.at[dstt], add=True)

    plsc.subcore_barrier()
    pltpu.sync_copy(acc.at[pl.ds(base, STRIPE1)], wb_v)
    pltpu.sync_copy(wb_v, out_hbm.at[c].at[pl.ds(base, STRIPE1)])


@functools.partial(
    pl.kernel,
    out_type=jax.ShapeDtypeStruct((2, NP, D_HID), jnp.float32),
    mesh=_SC_MESH,
    scratch_types=[
        [pltpu.VMEM((CH,), jnp.int32) for _ in range(2 * K)],
        [pltpu.VMEM((CH,), jnp.int32) for _ in range(2 * K)],
        [pltpu.VMEM((CH, D_HID), jnp.float32) for _ in range(K)],
        pltpu.VMEM((TAIL,), jnp.int32),
        pltpu.VMEM((TAIL,), jnp.int32),
        pltpu.VMEM((TAIL, D_HID), jnp.float32),
        pltpu.VMEM((STRIPE, D_HID), jnp.float32),
        pltpu.VMEM_SHARED((NP, D_HID), jnp.float32),
        [pltpu.SemaphoreType.DMA for _ in range(2)],
        [pltpu.SemaphoreType.DMA for _ in range(K)],
        [pltpu.SemaphoreType.DMA for _ in range(K)],
    ],
    compiler_params=_SC_PARAMS,
)
def _sc_aggregate(tab_hbm, src_hbm, dst_hbm, out_hbm, srcb, dstb, rows, srct, dstt,
                  rowst, wb_v, acc, si, sg, ss):
    c = lax.axis_index("c")
    s = lax.axis_index("s")
    base = s * STRIPE
    _zero_fill(wb_v, STRIPE)
    pltpu.sync_copy(wb_v, acc.at[pl.ds(base, STRIPE)])
    plsc.subcore_barrier()

    ebase = (c * 16 + s) * EPW

    def idx_at(row, g, b):
        ref = src_hbm if row == 0 else dst_hbm
        return ref.at[pl.ds(ebase + (g * K + b) * CH, CH)]

    for b in range(K):
        pltpu.async_copy(idx_at(0, 0, b), srcb[b], si[0])
        pltpu.async_copy(idx_at(1, 0, b), dstb[b], si[0])

    def phase(g, g2, p, first, last):
        for b in range(K):
            pltpu.make_async_copy(idx_at(0, g, b), srcb[p * K + b], si[p]).wait()
            pltpu.make_async_copy(idx_at(1, g, b), dstb[p * K + b], si[p]).wait()

        def wait_scatters():
            for b in range(K):
                pltpu.make_async_copy(rows[b], acc.at[dstb[p * K + b]],
                                      ss[b]).wait()

        if first:
            pl.when(g2 > 0)(wait_scatters)
        else:
            wait_scatters()

        for b in range(K):
            pltpu.async_copy(tab_hbm.at[srcb[p * K + b]], rows[b], sg[b])

        def prefetch():
            for b in range(K):
                pltpu.async_copy(idx_at(0, g + 1, b), srcb[(1 - p) * K + b],
                                 si[1 - p])
                pltpu.async_copy(idx_at(1, g + 1, b), dstb[(1 - p) * K + b],
                                 si[1 - p])

        if last:
            pl.when(g2 + 1 < G // 2)(prefetch)
        else:
            prefetch()

        for b in range(K):
            pltpu.make_async_copy(tab_hbm.at[srcb[p * K + b]], rows[b],
                                  sg[b]).wait()
            pltpu.async_copy(rows[b], acc.at[dstb[p * K + b]], ss[b], add=True)

    def step(g2, _):
        phase(2 * g2, g2, 0, True, False)
        phase(2 * g2 + 1, g2, 1, False, True)
        return 0

    lax.fori_loop(0, G // 2, step, 0)
    for b in range(K):
        pltpu.make_async_copy(rows[b], acc.at[dstb[b]], ss[b]).wait()

    pltpu.sync_copy(src_hbm.at[pl.ds(ebase + FULL * CH, TAIL)], srct)
    pltpu.sync_copy(dst_hbm.at[pl.ds(ebase + FULL * CH, TAIL)], dstt)
    pltpu.sync_copy(tab_hbm.at[srct], rowst)
    pltpu.sync_copy(rowst, acc.at[dstt], add=True)

    plsc.subcore_barrier()
    pltpu.sync_copy(acc.at[pl.ds(base, STRIPE)], wb_v)
    pltpu.sync_copy(wb_v, out_hbm.at[c].at[pl.ds(base, STRIPE)])


def _tc_dense1_body(x_ref, w1_ref, degp_ref, xw_ref, xs_ref, dinv_ref):
    xw = jnp.dot(x_ref[...], w1_ref[...], preferred_element_type=jnp.float32)
    deg = degp_ref[0, :N, :1] + degp_ref[1, :N, :1] + 1.0
    dinv = lax.rsqrt(deg)
    xw_ref[...] = xw
    xs_ref[...] = xw * dinv
    dinv_ref[...] = dinv


def _tc_dense2_body(ap_ref, xw_ref, dinv_ref, b1_ref, w2_ref, b2_ref, g_ref,
                    q_ref):
    a = ap_ref[0, :N] + ap_ref[1, :N]
    dinv = dinv_ref[...]
    h = jnp.maximum(dinv * a + dinv * dinv * xw_ref[...] + b1_ref[...], 0.0)
    w2 = w2_ref[...]
    wd = w2[:, :1] - w2[:, 1:2]
    t = jnp.sum(h * wd.T, axis=1, keepdims=True)
    g_ref[...] = t * dinv
    q_ref[...] = t * dinv * dinv + (b2_ref[0, 0] - b2_ref[0, 1])


def _tc_dense3_body(ap_ref, dinv_ref, q_ref, y_ref):
    a = ap_ref[0, :N, :1] + ap_ref[1, :N, :1]
    z = dinv_ref[...] * a + q_ref[...]
    sp_pos = jnp.maximum(z, 0.0) + jnp.log1p(jnp.exp(-jnp.abs(z)))
    sp_neg = sp_pos - z
    y_ref[...] = jnp.concatenate([-sp_neg, -sp_pos], axis=1)


def kernel(x, edge_index, W1, b1, W2, b2):
    src = edge_index[0]
    dst = edge_index[1]
    degp = _sc_degree(dst).reshape(2, NP1, 1)

    xw, xs, dinv = pl.pallas_call(
        _tc_dense1_body,
        out_shape=(
            jax.ShapeDtypeStruct((N, D_HID), jnp.float32),
            jax.ShapeDtypeStruct((N, D_HID), jnp.float32),
            jax.ShapeDtypeStruct((N, 1), jnp.float32),
        ),
    )(x, W1, degp)

    a1 = _sc_aggregate(xs, src, dst)

    g, q = pl.pallas_call(
        _tc_dense2_body,
        out_shape=(
            jax.ShapeDtypeStruct((N, 1), jnp.float32),
            jax.ShapeDtypeStruct((N, 1), jnp.float32),
        ),
    )(a1, xw, dinv, b1.reshape(1, D_HID), W2, b2.reshape(1, 2))

    a2 = _sc_aggregate1(g.reshape(N), src, dst).reshape(2, NP1, 1)

    y = pl.pallas_call(
        _tc_dense3_body,
        out_shape=jax.ShapeDtypeStruct((N, 2), jnp.float32),
    )(a2, dinv, q)

    return y


# speedup vs baseline: 1.1397x; 1.1397x over previous
import functools

import jax
import jax.numpy as jnp
from jax import lax
from jax.experimental import pallas as pl
from jax.experimental.pallas import tpu as pltpu
from jax.experimental.pallas import tpu_sc as plsc

N = 10000
D_IN = 128
D_HID = 16
E = 320000

NW = 32
EPW = E // NW
CH = 128
K = 3
FULL = EPW // CH
G = FULL // K
TAIL = EPW - FULL * CH
NP = 10112
STRIPE = NP // 16
NP1 = 10240
STRIPE1 = NP1 // 16

_SC_MESH = plsc.VectorSubcoreMesh(core_axis_name="c", subcore_axis_name="s")
_SC_PARAMS = pltpu.CompilerParams(use_tc_tiling_on_sc=False)


def _zero_fill(ref, nrows):
    def body(i, _):
        ref[i] = jnp.zeros((D_HID,), jnp.float32)
        return 0

    lax.fori_loop(0, nrows, body, 0)


@functools.partial(
    pl.kernel,
    out_type=jax.ShapeDtypeStruct((2, NP, D_HID), jnp.float32),
    mesh=_SC_MESH,
    scratch_types=[
        [pltpu.VMEM((CH,), jnp.int32) for _ in range(2 * K)],
        pltpu.VMEM((CH, D_HID), jnp.float32),
        pltpu.VMEM((STRIPE, D_HID), jnp.float32),
        pltpu.VMEM_SHARED((NP, D_HID), jnp.float32),
        [pltpu.SemaphoreType.DMA for _ in range(2)],
        [pltpu.SemaphoreType.DMA for _ in range(K)],
    ],
    compiler_params=_SC_PARAMS,
)
def _sc_degree(ei_hbm, out_hbm, dstb, ones_v, wb_v, acc, si, ss):
    c = lax.axis_index("c")
    s = lax.axis_index("s")
    base = s * STRIPE
    _zero_fill(wb_v, STRIPE)
    pltpu.sync_copy(wb_v, acc.at[pl.ds(base, STRIPE)])

    def fill_ones(i, _):
        ones_v[i] = jnp.ones((D_HID,), jnp.float32)
        return 0

    lax.fori_loop(0, CH, fill_ones, 0)
    plsc.subcore_barrier()

    ebase = (c * 16 + s) * EPW

    def idx_src(g, b):
        return ei_hbm.at[1, pl.ds(ebase + (g * K + b) * CH, CH)]

    for b in range(K):
        pltpu.async_copy(idx_src(0, b), dstb[b], si[0])

    def phase(g, g2, p, first, last):
        for b in range(K):
            pltpu.make_async_copy(idx_src(g, b), dstb[p * K + b], si[p]).wait()

        def wait_scatters():
            for b in range(K):
                pltpu.make_async_copy(ones_v, acc.at[dstb[p * K + b]],
                                      ss[b]).wait()

        if first:
            pl.when(g2 > 0)(wait_scatters)
        else:
            wait_scatters()

        for b in range(K):
            pltpu.async_copy(ones_v, acc.at[dstb[p * K + b]], ss[b], add=True)

        def prefetch():
            for b in range(K):
                pltpu.async_copy(idx_src(g + 1, b), dstb[(1 - p) * K + b],
                                 si[1 - p])

        if last:
            pl.when(g2 + 1 < G // 2)(prefetch)
        else:
            prefetch()

    def step(g2, _):
        phase(2 * g2, g2, 0, True, False)
        phase(2 * g2 + 1, g2, 1, False, True)
        return 0

    lax.fori_loop(0, G // 2, step, 0)
    for b in range(K):
        pltpu.make_async_copy(ones_v, acc.at[dstb[b]], ss[b]).wait()

    pltpu.sync_copy(ei_hbm.at[1, pl.ds(ebase + FULL * CH, TAIL)],
                    dstb[0].at[pl.ds(0, TAIL)])
    pltpu.sync_copy(ones_v.at[pl.ds(0, TAIL)],
                    acc.at[dstb[0].at[pl.ds(0, TAIL)]], add=True)

    plsc.subcore_barrier()
    pltpu.sync_copy(acc.at[pl.ds(base, STRIPE)], wb_v)
    pltpu.sync_copy(wb_v, out_hbm.at[c].at[pl.ds(base, STRIPE)])


@functools.partial(
    pl.kernel,
    out_type=jax.ShapeDtypeStruct((2, NP, D_HID), jnp.float32),
    mesh=_SC_MESH,
    scratch_types=[
        [pltpu.VMEM((CH,), jnp.int32) for _ in range(2 * K)],
        [pltpu.VMEM((CH,), jnp.int32) for _ in range(2 * K)],
        [pltpu.VMEM((CH, D_HID), jnp.float32) for _ in range(K)],
        pltpu.VMEM((TAIL,), jnp.int32),
        pltpu.VMEM((TAIL,), jnp.int32),
        pltpu.VMEM((TAIL, D_HID), jnp.float32),
        pltpu.VMEM((STRIPE, D_HID), jnp.float32),
        pltpu.VMEM_SHARED((NP, D_HID), jnp.float32),
        [pltpu.SemaphoreType.DMA for _ in range(2)],
        [pltpu.SemaphoreType.DMA for _ in range(K)],
        [pltpu.SemaphoreType.DMA for _ in range(K)],
    ],
    compiler_params=_SC_PARAMS,
)
def _sc_aggregate(tab_hbm, ei_hbm, out_hbm, srcb, dstb, rows, srct, dstt,
                  rowst, wb_v, acc, si, sg, ss):
    c = lax.axis_index("c")
    s = lax.axis_index("s")
    base = s * STRIPE
    _zero_fill(wb_v, STRIPE)
    pltpu.sync_copy(wb_v, acc.at[pl.ds(base, STRIPE)])
    plsc.subcore_barrier()

    ebase = (c * 16 + s) * EPW

    def idx_at(row, g, b):
        return ei_hbm.at[row, pl.ds(ebase + (g * K + b) * CH, CH)]

    for b in range(K):
        pltpu.async_copy(idx_at(0, 0, b), srcb[b], si[0])
        pltpu.async_copy(idx_at(1, 0, b), dstb[b], si[0])

    def phase(g, g2, p, first, last):
        for b in range(K):
            pltpu.make_async_copy(idx_at(0, g, b), srcb[p * K + b], si[p]).wait()
            pltpu.make_async_copy(idx_at(1, g, b), dstb[p * K + b], si[p]).wait()

        def wait_scatters():
            for b in range(K):
                pltpu.make_async_copy(rows[b], acc.at[dstb[p * K + b]],
                                      ss[b]).wait()

        if first:
            pl.when(g2 > 0)(wait_scatters)
        else:
            wait_scatters()

        for b in range(K):
            pltpu.async_copy(tab_hbm.at[srcb[p * K + b]], rows[b], sg[b])

        def prefetch():
            for b in range(K):
                pltpu.async_copy(idx_at(0, g + 1, b), srcb[(1 - p) * K + b],
                                 si[1 - p])
                pltpu.async_copy(idx_at(1, g + 1, b), dstb[(1 - p) * K + b],
                                 si[1 - p])

        if last:
            pl.when(g2 + 1 < G // 2)(prefetch)
        else:
            prefetch()

        for b in range(K):
            pltpu.make_async_copy(tab_hbm.at[srcb[p * K + b]], rows[b],
                                  sg[b]).wait()
            pltpu.async_copy(rows[b], acc.at[dstb[p * K + b]], ss[b], add=True)

    def step(g2, _):
        phase(2 * g2, g2, 0, True, False)
        phase(2 * g2 + 1, g2, 1, False, True)
        return 0

    lax.fori_loop(0, G // 2, step, 0)
    for b in range(K):
        pltpu.make_async_copy(rows[b], acc.at[dstb[b]], ss[b]).wait()

    pltpu.sync_copy(ei_hbm.at[0, pl.ds(ebase + FULL * CH, TAIL)], srct)
    pltpu.sync_copy(ei_hbm.at[1, pl.ds(ebase + FULL * CH, TAIL)], dstt)
    pltpu.sync_copy(tab_hbm.at[srct], rowst)
    pltpu.sync_copy(rowst, acc.at[dstt], add=True)

    plsc.subcore_barrier()
    pltpu.sync_copy(acc.at[pl.ds(base, STRIPE)], wb_v)
    pltpu.sync_copy(wb_v, out_hbm.at[c].at[pl.ds(base, STRIPE)])


def _tc_dense1_body(x_ref, w1_ref, degp_ref, xw_ref, xs_ref, dinv_ref):
    xw = jnp.dot(x_ref[...], w1_ref[...], preferred_element_type=jnp.float32)
    deg = degp_ref[0, :N] + degp_ref[1, :N] + 1.0
    dinv = lax.rsqrt(deg)
    xw_ref[...] = xw
    xs_ref[...] = xw * dinv
    dinv_ref[...] = dinv


def _tc_dense2_body(ap_ref, xw_ref, dinv_ref, b1_ref, w2_ref, b2_ref, g_ref,
                    q_ref):
    a = ap_ref[0, :N] + ap_ref[1, :N]
    dinv = dinv_ref[...]
    h = jnp.maximum(dinv * a + dinv * dinv * xw_ref[...] + b1_ref[...], 0.0)
    w2 = w2_ref[...]
    wd = w2[:, :1] - w2[:, 1:2]
    t = jnp.sum(h * wd.T, axis=1, keepdims=True)
    g_ref[...] = t * dinv
    q_ref[...] = t * dinv * dinv + (b2_ref[0, 0] - b2_ref[0, 1])


def _tc_dense3_body(ap_ref, dinv_ref, q_ref, y_ref):
    a = ap_ref[0, :N] + ap_ref[1, :N]
    z = dinv_ref[...] * a + q_ref[...]
    sp_pos = jnp.maximum(z, 0.0) + jnp.log1p(jnp.exp(-jnp.abs(z)))
    sp_neg = sp_pos - z
    y_ref[...] = jnp.concatenate([-sp_neg[:, :1], -sp_pos[:, :1]], axis=1)


def kernel(x, edge_index, W1, b1, W2, b2):
    degp = _sc_degree(edge_index)

    xw, xs, dinv = pl.pallas_call(
        _tc_dense1_body,
        out_shape=(
            jax.ShapeDtypeStruct((N, D_HID), jnp.float32),
            jax.ShapeDtypeStruct((N, D_HID), jnp.float32),
            jax.ShapeDtypeStruct((N, D_HID), jnp.float32),
        ),
    )(x, W1, degp)

    a1 = _sc_aggregate(xs, edge_index)

    g, q = pl.pallas_call(
        _tc_dense2_body,
        out_shape=(
            jax.ShapeDtypeStruct((N, D_HID), jnp.float32),
            jax.ShapeDtypeStruct((N, D_HID), jnp.float32),
        ),
    )(a1, xw, dinv, b1.reshape(1, D_HID), W2, b2.reshape(1, 2))

    a2 = _sc_aggregate(g, edge_index)

    y = pl.pallas_call(
        _tc_dense3_body,
        out_shape=jax.ShapeDtypeStruct((N, 2), jnp.float32),
    )(a2, dinv, q)

    return y


# speedup vs baseline: 1.5902x; 1.3953x over previous
import functools

import jax
import jax.numpy as jnp
from jax import lax
from jax.experimental import pallas as pl
from jax.experimental.pallas import tpu as pltpu
from jax.experimental.pallas import tpu_sc as plsc

N = 10000
D_IN = 128
D_HID = 16
E = 320000

NW = 32
EPW = E // NW
CH = 128
K = 3
FULL = EPW // CH
G = FULL // K
TAIL = EPW - FULL * CH
NP = 10112
STRIPE = NP // 16
NP1 = 10240
STRIPE1 = NP1 // 16

_SC_MESH = plsc.VectorSubcoreMesh(core_axis_name="c", subcore_axis_name="s")
_SC_PARAMS = pltpu.CompilerParams(use_tc_tiling_on_sc=False)


def _zero_fill(ref, nrows):
    def body(i, _):
        ref[i] = jnp.zeros((D_HID,), jnp.float32)
        return 0

    lax.fori_loop(0, nrows, body, 0)


@functools.partial(
    pl.kernel,
    out_type=jax.ShapeDtypeStruct((2, NP, D_HID), jnp.float32),
    mesh=_SC_MESH,
    scratch_types=[
        [pltpu.VMEM((CH,), jnp.int32) for _ in range(2 * K)],
        pltpu.VMEM((CH, D_HID), jnp.float32),
        pltpu.VMEM((STRIPE, D_HID), jnp.float32),
        pltpu.VMEM_SHARED((NP, D_HID), jnp.float32),
        [pltpu.SemaphoreType.DMA for _ in range(2)],
        [pltpu.SemaphoreType.DMA for _ in range(K)],
    ],
    compiler_params=_SC_PARAMS,
)
def _sc_degree(ei_hbm, out_hbm, dstb, ones_v, wb_v, acc, si, ss):
    c = lax.axis_index("c")
    s = lax.axis_index("s")
    base = s * STRIPE
    _zero_fill(wb_v, STRIPE)
    pltpu.sync_copy(wb_v, acc.at[pl.ds(base, STRIPE)])

    def fill_ones(i, _):
        ones_v[i] = jnp.ones((D_HID,), jnp.float32)
        return 0

    lax.fori_loop(0, CH, fill_ones, 0)
    plsc.subcore_barrier()

    ebase = (c * 16 + s) * EPW

    def idx_src(g, b):
        return ei_hbm.at[1, pl.ds(ebase + (g * K + b) * CH, CH)]

    for b in range(K):
        pltpu.async_copy(idx_src(0, b), dstb[b], si[0])

    def phase(g, g2, p, first, last):
        for b in range(K):
            pltpu.make_async_copy(idx_src(g, b), dstb[p * K + b], si[p]).wait()

        def wait_scatters():
            for b in range(K):
                pltpu.make_async_copy(ones_v, acc.at[dstb[p * K + b]],
                                      ss[b]).wait()

        if first:
            pl.when(g2 > 0)(wait_scatters)
        else:
            wait_scatters()

        for b in range(K):
            pltpu.async_copy(ones_v, acc.at[dstb[p * K + b]], ss[b], add=True)

        def prefetch():
            for b in range(K):
                pltpu.async_copy(idx_src(g + 1, b), dstb[(1 - p) * K + b],
                                 si[1 - p])

        if last:
            pl.when(g2 + 1 < G // 2)(prefetch)
        else:
            prefetch()

    def step(g2, _):
        phase(2 * g2, g2, 0, True, False)
        phase(2 * g2 + 1, g2, 1, False, True)
        return 0

    lax.fori_loop(0, G // 2, step, 0)
    for b in range(K):
        pltpu.make_async_copy(ones_v, acc.at[dstb[b]], ss[b]).wait()

    pltpu.sync_copy(ei_hbm.at[1, pl.ds(ebase + FULL * CH, TAIL)],
                    dstb[0].at[pl.ds(0, TAIL)])
    pltpu.sync_copy(ones_v.at[pl.ds(0, TAIL)],
                    acc.at[dstb[0].at[pl.ds(0, TAIL)]], add=True)

    plsc.subcore_barrier()
    pltpu.sync_copy(acc.at[pl.ds(base, STRIPE)], wb_v)
    pltpu.sync_copy(wb_v, out_hbm.at[c].at[pl.ds(base, STRIPE)])


@functools.partial(
    pl.kernel,
    out_type=jax.ShapeDtypeStruct((2, NP, D_HID), jnp.float32),
    mesh=_SC_MESH,
    scratch_types=[
        [pltpu.VMEM((CH,), jnp.int32) for _ in range(2 * K)],
        [pltpu.VMEM((CH,), jnp.int32) for _ in range(2 * K)],
        [pltpu.VMEM((CH, D_HID), jnp.float32) for _ in range(K)],
        pltpu.VMEM((TAIL,), jnp.int32),
        pltpu.VMEM((TAIL,), jnp.int32),
        pltpu.VMEM((TAIL, D_HID), jnp.float32),
        pltpu.VMEM((STRIPE, D_HID), jnp.float32),
        pltpu.VMEM_SHARED((NP, D_HID), jnp.float32),
        [pltpu.SemaphoreType.DMA for _ in range(2)],
        [pltpu.SemaphoreType.DMA for _ in range(K)],
        [pltpu.SemaphoreType.DMA for _ in range(K)],
    ],
    compiler_params=_SC_PARAMS,
)
def _sc_aggregate(tab_hbm, ei_hbm, out_hbm, srcb, dstb, rows, srct, dstt,
                  rowst, wb_v, acc, si, sg, ss):
    c = lax.axis_index("c")
    s = lax.axis_index("s")
    base = s * STRIPE
    _zero_fill(wb_v, STRIPE)
    pltpu.sync_copy(wb_v, acc.at[pl.ds(base, STRIPE)])
    plsc.subcore_barrier()

    ebase = (c * 16 + s) * EPW

    def idx_at(row, g, b):
        return ei_hbm.at[row, pl.ds(ebase + (g * K + b) * CH, CH)]

    for b in range(K):
        pltpu.async_copy(idx_at(0, 0, b), srcb[b], si[0])
        pltpu.async_copy(idx_at(1, 0, b), dstb[b], si[0])

    def phase(g, g2, p, first, last):
        for b in range(K):
            pltpu.make_async_copy(idx_at(0, g, b), srcb[p * K + b], si[p]).wait()
            pltpu.make_async_copy(idx_at(1, g, b), dstb[p * K + b], si[p]).wait()

        def wait_scatters():
            for b in range(K):
                pltpu.make_async_copy(rows[b], acc.at[dstb[p * K + b]],
                                      ss[b]).wait()

        if first:
            pl.when(g2 > 0)(wait_scatters)
        else:
            wait_scatters()

        for b in range(K):
            pltpu.async_copy(tab_hbm.at[srcb[p * K + b]], rows[b], sg[b])

        def prefetch():
            for b in range(K):
                pltpu.async_copy(idx_at(0, g + 1, b), srcb[(1 - p) * K + b],
                                 si[1 - p])
                pltpu.async_copy(idx_at(1, g + 1, b), dstb[(1 - p) * K + b],
                                 si[1 - p])

        if last:
            pl.when(g2 + 1 < G // 2)(prefetch)
        else:
            prefetch()

        for b in range(K):
            pltpu.make_async_copy(tab_hbm.at[srcb[p * K + b]], rows[b],
                                  sg[b]).wait()
            pltpu.async_copy(rows[b], acc.at[dstb[p * K + b]], ss[b], add=True)

    def step(g2, _):
        phase(2 * g2, g2, 0, True, False)
        phase(2 * g2 + 1, g2, 1, False, True)
        return 0

    lax.fori_loop(0, G // 2, step, 0)
    for b in range(K):
        pltpu.make_async_copy(rows[b], acc.at[dstb[b]], ss[b]).wait()

    pltpu.sync_copy(ei_hbm.at[0, pl.ds(ebase + FULL * CH, TAIL)], srct)
    pltpu.sync_copy(ei_hbm.at[1, pl.ds(ebase + FULL * CH, TAIL)], dstt)
    pltpu.sync_copy(tab_hbm.at[srct], rowst)
    pltpu.sync_copy(rowst, acc.at[dstt], add=True)

    plsc.subcore_barrier()
    pltpu.sync_copy(acc.at[pl.ds(base, STRIPE)], wb_v)
    pltpu.sync_copy(wb_v, out_hbm.at[c].at[pl.ds(base, STRIPE)])


NPK = N // 8
NPP = NP // 8


def _group_iota(shape, d0, d1):
    a = lax.broadcasted_iota(jnp.int32, shape, d0) // 16
    b = lax.broadcasted_iota(jnp.int32, shape, d1)
    return (a == b).astype(jnp.float32)


def _tc_dense1_body(x_ref, w1_ref, degp_ref, xs_ref, dinv_ref):
    w1 = w1_ref[...]
    w1t = jnp.tile(w1, (8, 8))
    r = lax.broadcasted_iota(jnp.int32, (8 * D_IN, 128), 0) // D_IN
    c = lax.broadcasted_iota(jnp.int32, (8 * D_IN, 128), 1) // D_HID
    w1bd = w1t * (r == c).astype(jnp.float32)
    xw_p = jnp.dot(x_ref[...], w1bd, preferred_element_type=jnp.float32)
    deg = degp_ref[0, :NPK] + degp_ref[1, :NPK] + 1.0
    dinv = lax.rsqrt(deg)
    xs_ref[...] = xw_p * dinv
    dinv_ref[...] = dinv


def _tc_dense2_body(ap_ref, xs_ref, dinv_ref, b1_ref, w2_ref, b2_ref, g_ref,
                    q_ref):
    a = ap_ref[0, :NPK] + ap_ref[1, :NPK]
    dinv = dinv_ref[...]
    h = jnp.maximum(dinv * a + dinv * xs_ref[...] + b1_ref[...], 0.0)
    w2 = w2_ref[...]
    wd = w2[:, 0] - w2[:, 1]
    wd128 = jnp.concatenate([wd] * 8)
    hw = h * wd128
    t8 = jnp.dot(hw, _group_iota((128, 8), 0, 1),
                 preferred_element_type=jnp.float32)
    t = jnp.dot(t8, _group_iota((8, 128), 1, 0),
                preferred_element_type=jnp.float32)
    g_ref[...] = t * dinv
    q_ref[...] = t * dinv * dinv + (b2_ref[0, 0] - b2_ref[0, 1])


def _tc_dense3_body(ap_ref, dinv_ref, q_ref, y_ref):
    a = ap_ref[0, :NPK] + ap_ref[1, :NPK]
    z = dinv_ref[...] * a + q_ref[...]
    sp_pos = jnp.maximum(z, 0.0) + jnp.log1p(jnp.exp(-jnp.abs(z)))
    sp_neg = sp_pos - z
    pick = _group_iota((128, 8), 0, 1) * (
        (lax.broadcasted_iota(jnp.int32, (128, 8), 0) % 16 == 0)
        .astype(jnp.float32))
    y0 = jnp.dot(-sp_neg, pick, preferred_element_type=jnp.float32)
    y1 = jnp.dot(-sp_pos, pick, preferred_element_type=jnp.float32)
    i0 = lax.broadcasted_iota(jnp.int32, (8, 16), 0)
    i1 = lax.broadcasted_iota(jnp.int32, (8, 16), 1)
    p0 = (i1 == 2 * i0).astype(jnp.float32)
    p1 = (i1 == 2 * i0 + 1).astype(jnp.float32)
    y_ref[...] = (jnp.dot(y0, p0, preferred_element_type=jnp.float32)
                  + jnp.dot(y1, p1, preferred_element_type=jnp.float32))


def kernel(x, edge_index, W1, b1, W2, b2):
    degp = _sc_degree(edge_index).reshape(2, NPP, 128)

    xs, dinv = pl.pallas_call(
        _tc_dense1_body,
        out_shape=(
            jax.ShapeDtypeStruct((NPK, 128), jnp.float32),
            jax.ShapeDtypeStruct((NPK, 128), jnp.float32),
        ),
    )(x.reshape(NPK, 8 * D_IN), W1, degp)

    a1 = _sc_aggregate(xs.reshape(N, D_HID), edge_index).reshape(2, NPP, 128)

    b1_128 = jnp.tile(b1.reshape(1, D_HID), (1, 8))
    g, q = pl.pallas_call(
        _tc_dense2_body,
        out_shape=(
            jax.ShapeDtypeStruct((NPK, 128), jnp.float32),
            jax.ShapeDtypeStruct((NPK, 128), jnp.float32),
        ),
    )(a1, xs, dinv, b1_128, W2, b2.reshape(1, 2))

    a2 = _sc_aggregate(g.reshape(N, D_HID), edge_index).reshape(2, NPP, 128)

    yp = pl.pallas_call(
        _tc_dense3_body,
        out_shape=jax.ShapeDtypeStruct((NPK, 16), jnp.float32),
    )(a2, dinv, q)

    return yp.reshape(N, 2)


# speedup vs baseline: 1.7936x; 1.1279x over previous
import functools

import jax
import jax.numpy as jnp
from jax import lax
from jax.experimental import pallas as pl
from jax.experimental.pallas import tpu as pltpu
from jax.experimental.pallas import tpu_sc as plsc

N = 10000
D_IN = 128
D_HID = 16
E = 320000

NW = 32
EPW = E // NW
CH = 128
K = 3
FULL = EPW // CH
G = FULL // K
TAIL = EPW - FULL * CH
NP = 10112
STRIPE = NP // 16
NP1 = 10240
STRIPE1 = NP1 // 16

_SC_MESH = plsc.VectorSubcoreMesh(core_axis_name="c", subcore_axis_name="s")
_SC_PARAMS = pltpu.CompilerParams(use_tc_tiling_on_sc=False)


def _zero_fill(ref, nrows):
    def body(i, _):
        ref[i] = jnp.zeros((D_HID,), jnp.float32)
        return 0

    lax.fori_loop(0, nrows, body, 0)


@functools.partial(
    pl.kernel,
    out_type=jax.ShapeDtypeStruct((2, NP, D_HID), jnp.float32),
    mesh=_SC_MESH,
    scratch_types=[
        [pltpu.VMEM((CH,), jnp.int32) for _ in range(2 * K)],
        pltpu.VMEM((CH, D_HID), jnp.float32),
        pltpu.VMEM((STRIPE, D_HID), jnp.float32),
        pltpu.VMEM_SHARED((NP, D_HID), jnp.float32),
        [pltpu.SemaphoreType.DMA for _ in range(2)],
        [pltpu.SemaphoreType.DMA for _ in range(K)],
    ],
    compiler_params=_SC_PARAMS,
)
def _sc_degree(ei_hbm, out_hbm, dstb, ones_v, wb_v, acc, si, ss):
    c = lax.axis_index("c")
    s = lax.axis_index("s")
    base = s * STRIPE
    _zero_fill(wb_v, STRIPE)
    pltpu.sync_copy(wb_v, acc.at[pl.ds(base, STRIPE)])

    def fill_ones(i, _):
        ones_v[i] = jnp.ones((D_HID,), jnp.float32)
        return 0

    lax.fori_loop(0, CH, fill_ones, 0)
    plsc.subcore_barrier()

    ebase = (c * 16 + s) * EPW

    def idx_src(g, b):
        return ei_hbm.at[1, pl.ds(ebase + (g * K + b) * CH, CH)]

    for b in range(K):
        pltpu.async_copy(idx_src(0, b), dstb[b], si[0])

    def phase(g, g2, p, first, last):
        for b in range(K):
            pltpu.make_async_copy(idx_src(g, b), dstb[p * K + b], si[p]).wait()

        def wait_scatters():
            for b in range(K):
                pltpu.make_async_copy(ones_v, acc.at[dstb[p * K + b]],
                                      ss[b]).wait()

        if first:
            pl.when(g2 > 0)(wait_scatters)
        else:
            wait_scatters()

        for b in range(K):
            pltpu.async_copy(ones_v, acc.at[dstb[p * K + b]], ss[b], add=True)

        def prefetch():
            for b in range(K):
                pltpu.async_copy(idx_src(g + 1, b), dstb[(1 - p) * K + b],
                                 si[1 - p])

        if last:
            pl.when(g2 + 1 < G // 2)(prefetch)
        else:
            prefetch()

    def step(g2, _):
        phase(2 * g2, g2, 0, True, False)
        phase(2 * g2 + 1, g2, 1, False, True)
        return 0

    lax.fori_loop(0, G // 2, step, 0)
    for b in range(K):
        pltpu.make_async_copy(ones_v, acc.at[dstb[b]], ss[b]).wait()

    pltpu.sync_copy(ei_hbm.at[1, pl.ds(ebase + FULL * CH, TAIL)],
                    dstb[0].at[pl.ds(0, TAIL)])
    pltpu.sync_copy(ones_v.at[pl.ds(0, TAIL)],
                    acc.at[dstb[0].at[pl.ds(0, TAIL)]], add=True)

    plsc.subcore_barrier()
    pltpu.sync_copy(acc.at[pl.ds(base, STRIPE)], wb_v)
    pltpu.sync_copy(wb_v, out_hbm.at[c].at[pl.ds(base, STRIPE)])


@functools.partial(
    pl.kernel,
    out_type=jax.ShapeDtypeStruct((2, NP, D_HID), jnp.float32),
    mesh=_SC_MESH,
    scratch_types=[
        [pltpu.VMEM((CH,), jnp.int32) for _ in range(2 * K)],
        [pltpu.VMEM((CH,), jnp.int32) for _ in range(2 * K)],
        [pltpu.VMEM((CH, D_HID), jnp.float32) for _ in range(K)],
        pltpu.VMEM((TAIL,), jnp.int32),
        pltpu.VMEM((TAIL,), jnp.int32),
        pltpu.VMEM((TAIL, D_HID), jnp.float32),
        pltpu.VMEM((STRIPE, D_HID), jnp.float32),
        pltpu.VMEM_SHARED((NP, D_HID), jnp.float32),
        pltpu.VMEM((N // 16, D_HID), jnp.float32),
        pltpu.VMEM_SHARED((N, D_HID), jnp.float32),
        [pltpu.SemaphoreType.DMA for _ in range(2)],
        [pltpu.SemaphoreType.DMA for _ in range(K)],
        [pltpu.SemaphoreType.DMA for _ in range(K)],
    ],
    compiler_params=_SC_PARAMS,
)
def _sc_aggregate(tab_hbm, ei_hbm, out_hbm, srcb, dstb, rows, srct, dstt,
                  rowst, wb_v, acc, tstage, tab_sp, si, sg, ss):
    c = lax.axis_index("c")
    s = lax.axis_index("s")
    base = s * STRIPE
    _zero_fill(wb_v, STRIPE)
    pltpu.sync_copy(wb_v, acc.at[pl.ds(base, STRIPE)])
    tbase = s * (N // 16)
    pltpu.sync_copy(tab_hbm.at[pl.ds(tbase, N // 16)], tstage)
    pltpu.sync_copy(tstage, tab_sp.at[pl.ds(tbase, N // 16)])
    plsc.subcore_barrier()

    ebase = (c * 16 + s) * EPW

    def idx_at(row, g, b):
        return ei_hbm.at[row, pl.ds(ebase + (g * K + b) * CH, CH)]

    for b in range(K):
        pltpu.async_copy(idx_at(0, 0, b), srcb[b], si[0])
        pltpu.async_copy(idx_at(1, 0, b), dstb[b], si[0])

    def phase(g, g2, p, first, last):
        for b in range(K):
            pltpu.make_async_copy(idx_at(0, g, b), srcb[p * K + b], si[p]).wait()
            pltpu.make_async_copy(idx_at(1, g, b), dstb[p * K + b], si[p]).wait()

        def wait_scatters():
            for b in range(K):
                pltpu.make_async_copy(rows[b], acc.at[dstb[p * K + b]],
                                      ss[b]).wait()

        if first:
            pl.when(g2 > 0)(wait_scatters)
        else:
            wait_scatters()

        for b in range(K):
            pltpu.async_copy(tab_sp.at[srcb[p * K + b]], rows[b], sg[b])

        def prefetch():
            for b in range(K):
                pltpu.async_copy(idx_at(0, g + 1, b), srcb[(1 - p) * K + b],
                                 si[1 - p])
                pltpu.async_copy(idx_at(1, g + 1, b), dstb[(1 - p) * K + b],
                                 si[1 - p])

        if last:
            pl.when(g2 + 1 < G // 2)(prefetch)
        else:
            prefetch()

        for b in range(K):
            pltpu.make_async_copy(tab_sp.at[srcb[p * K + b]], rows[b],
                                  sg[b]).wait()
            pltpu.async_copy(rows[b], acc.at[dstb[p * K + b]], ss[b], add=True)

    def step(g2, _):
        phase(2 * g2, g2, 0, True, False)
        phase(2 * g2 + 1, g2, 1, False, True)
        return 0

    lax.fori_loop(0, G // 2, step, 0)
    for b in range(K):
        pltpu.make_async_copy(rows[b], acc.at[dstb[b]], ss[b]).wait()

    pltpu.sync_copy(ei_hbm.at[0, pl.ds(ebase + FULL * CH, TAIL)], srct)
    pltpu.sync_copy(ei_hbm.at[1, pl.ds(ebase + FULL * CH, TAIL)], dstt)
    pltpu.sync_copy(tab_sp.at[srct], rowst)
    pltpu.sync_copy(rowst, acc.at[dstt], add=True)

    plsc.subcore_barrier()
    pltpu.sync_copy(acc.at[pl.ds(base, STRIPE)], wb_v)
    pltpu.sync_copy(wb_v, out_hbm.at[c].at[pl.ds(base, STRIPE)])


NPK = N // 8
NPP = NP // 8


def _group_iota(shape, d0, d1):
    a = lax.broadcasted_iota(jnp.int32, shape, d0) // 16
    b = lax.broadcasted_iota(jnp.int32, shape, d1)
    return (a == b).astype(jnp.float32)


def _tc_dense1_body(x_ref, w1_ref, degp_ref, xs_ref, dinv_ref):
    w1 = w1_ref[...]
    w1t = jnp.tile(w1, (8, 8))
    r = lax.broadcasted_iota(jnp.int32, (8 * D_IN, 128), 0) // D_IN
    c = lax.broadcasted_iota(jnp.int32, (8 * D_IN, 128), 1) // D_HID
    w1bd = w1t * (r == c).astype(jnp.float32)
    xw_p = jnp.dot(x_ref[...], w1bd, preferred_element_type=jnp.float32)
    deg = degp_ref[0, :NPK] + degp_ref[1, :NPK] + 1.0
    dinv = lax.rsqrt(deg)
    xs_ref[...] = xw_p * dinv
    dinv_ref[...] = dinv


def _tc_dense2_body(ap_ref, xs_ref, dinv_ref, b1_ref, w2_ref, b2_ref, g_ref,
                    q_ref):
    a = ap_ref[0, :NPK] + ap_ref[1, :NPK]
    dinv = dinv_ref[...]
    h = jnp.maximum(dinv * a + dinv * xs_ref[...] + b1_ref[...], 0.0)
    w2 = w2_ref[...]
    wd = w2[:, 0] - w2[:, 1]
    wd128 = jnp.concatenate([wd] * 8)
    hw = h * wd128
    t8 = jnp.dot(hw, _group_iota((128, 8), 0, 1),
                 preferred_element_type=jnp.float32)
    t = jnp.dot(t8, _group_iota((8, 128), 1, 0),
                preferred_element_type=jnp.float32)
    g_ref[...] = t * dinv
    q_ref[...] = t * dinv * dinv + (b2_ref[0, 0] - b2_ref[0, 1])


def _tc_dense3_body(ap_ref, dinv_ref, q_ref, y_ref):
    a = ap_ref[0, :NPK] + ap_ref[1, :NPK]
    z = dinv_ref[...] * a + q_ref[...]
    sp_pos = jnp.maximum(z, 0.0) + jnp.log1p(jnp.exp(-jnp.abs(z)))
    sp_neg = sp_pos - z
    pick = _group_iota((128, 8), 0, 1) * (
        (lax.broadcasted_iota(jnp.int32, (128, 8), 0) % 16 == 0)
        .astype(jnp.float32))
    y0 = jnp.dot(-sp_neg, pick, preferred_element_type=jnp.float32)
    y1 = jnp.dot(-sp_pos, pick, preferred_element_type=jnp.float32)
    i0 = lax.broadcasted_iota(jnp.int32, (8, 16), 0)
    i1 = lax.broadcasted_iota(jnp.int32, (8, 16), 1)
    p0 = (i1 == 2 * i0).astype(jnp.float32)
    p1 = (i1 == 2 * i0 + 1).astype(jnp.float32)
    y_ref[...] = (jnp.dot(y0, p0, preferred_element_type=jnp.float32)
                  + jnp.dot(y1, p1, preferred_element_type=jnp.float32))


def kernel(x, edge_index, W1, b1, W2, b2):
    degp = _sc_degree(edge_index).reshape(2, NPP, 128)

    xs, dinv = pl.pallas_call(
        _tc_dense1_body,
        out_shape=(
            jax.ShapeDtypeStruct((NPK, 128), jnp.float32),
            jax.ShapeDtypeStruct((NPK, 128), jnp.float32),
        ),
    )(x.reshape(NPK, 8 * D_IN), W1, degp)

    a1 = _sc_aggregate(xs.reshape(N, D_HID), edge_index).reshape(2, NPP, 128)

    b1_128 = jnp.tile(b1.reshape(1, D_HID), (1, 8))
    g, q = pl.pallas_call(
        _tc_dense2_body,
        out_shape=(
            jax.ShapeDtypeStruct((NPK, 128), jnp.float32),
            jax.ShapeDtypeStruct((NPK, 128), jnp.float32),
        ),
    )(a1, xs, dinv, b1_128, W2, b2.reshape(1, 2))

    a2 = _sc_aggregate(g.reshape(N, D_HID), edge_index).reshape(2, NPP, 128)

    yp = pl.pallas_call(
        _tc_dense3_body,
        out_shape=jax.ShapeDtypeStruct((NPK, 16), jnp.float32),
    )(a2, dinv, q)

    return yp.reshape(N, 2)


# speedup vs baseline: 2.0404x; 1.1376x over previous
import functools

import jax
import jax.numpy as jnp
from jax import lax
from jax.experimental import pallas as pl
from jax.experimental.pallas import tpu as pltpu
from jax.experimental.pallas import tpu_sc as plsc

N = 10000
D_IN = 128
D_HID = 16
E = 320000

NW = 32
EPW = E // NW
CH = 128
K = 3
KA = 6
FULL = EPW // CH
G = FULL // K
GA = FULL // KA
TAIL = EPW - FULL * CH
NP = 10112
STRIPE = NP // 16
NP1 = 10240
STRIPE1 = NP1 // 16

_SC_MESH = plsc.VectorSubcoreMesh(core_axis_name="c", subcore_axis_name="s")
_SC_PARAMS = pltpu.CompilerParams(use_tc_tiling_on_sc=False)


def _zero_fill(ref, nrows):
    def body(i, _):
        ref[i] = jnp.zeros((D_HID,), jnp.float32)
        return 0

    lax.fori_loop(0, nrows, body, 0)


@functools.partial(
    pl.kernel,
    out_type=jax.ShapeDtypeStruct((2, NP, D_HID), jnp.float32),
    mesh=_SC_MESH,
    scratch_types=[
        [pltpu.VMEM((CH,), jnp.int32) for _ in range(2 * K)],
        pltpu.VMEM((CH, D_HID), jnp.float32),
        pltpu.VMEM((STRIPE, D_HID), jnp.float32),
        pltpu.VMEM_SHARED((NP, D_HID), jnp.float32),
        [pltpu.SemaphoreType.DMA for _ in range(2)],
        [pltpu.SemaphoreType.DMA for _ in range(K)],
    ],
    compiler_params=_SC_PARAMS,
)
def _sc_degree(ei_hbm, out_hbm, dstb, ones_v, wb_v, acc, si, ss):
    c = lax.axis_index("c")
    s = lax.axis_index("s")
    base = s * STRIPE
    _zero_fill(wb_v, STRIPE)
    pltpu.sync_copy(wb_v, acc.at[pl.ds(base, STRIPE)])

    def fill_ones(i, _):
        ones_v[i] = jnp.ones((D_HID,), jnp.float32)
        return 0

    lax.fori_loop(0, CH, fill_ones, 0)
    plsc.subcore_barrier()

    ebase = (c * 16 + s) * EPW

    def idx_src(g, b):
        return ei_hbm.at[1, pl.ds(ebase + (g * K + b) * CH, CH)]

    for b in range(K):
        pltpu.async_copy(idx_src(0, b), dstb[b], si[0])

    def phase(g, g2, p, first, last):
        for b in range(K):
            pltpu.make_async_copy(idx_src(g, b), dstb[p * K + b], si[p]).wait()

        def wait_scatters():
            for b in range(K):
                pltpu.make_async_copy(ones_v, acc.at[dstb[p * K + b]],
                                      ss[b]).wait()

        if first:
            pl.when(g2 > 0)(wait_scatters)
        else:
            wait_scatters()

        for b in range(K):
            pltpu.async_copy(ones_v, acc.at[dstb[p * K + b]], ss[b], add=True)

        def prefetch():
            for b in range(K):
                pltpu.async_copy(idx_src(g + 1, b), dstb[(1 - p) * K + b],
                                 si[1 - p])

        if last:
            pl.when(g2 + 1 < G // 2)(prefetch)
        else:
            prefetch()

    def step(g2, _):
        phase(2 * g2, g2, 0, True, False)
        phase(2 * g2 + 1, g2, 1, False, True)
        return 0

    lax.fori_loop(0, G // 2, step, 0)
    for b in range(K):
        pltpu.make_async_copy(ones_v, acc.at[dstb[b]], ss[b]).wait()

    pltpu.sync_copy(ei_hbm.at[1, pl.ds(ebase + FULL * CH, TAIL)],
                    dstb[0].at[pl.ds(0, TAIL)])
    pltpu.sync_copy(ones_v.at[pl.ds(0, TAIL)],
                    acc.at[dstb[0].at[pl.ds(0, TAIL)]], add=True)

    plsc.subcore_barrier()
    pltpu.sync_copy(acc.at[pl.ds(base, STRIPE)], wb_v)
    pltpu.sync_copy(wb_v, out_hbm.at[c].at[pl.ds(base, STRIPE)])


@functools.partial(
    pl.kernel,
    out_type=jax.ShapeDtypeStruct((2, NP, D_HID), jnp.float32),
    mesh=_SC_MESH,
    scratch_types=[
        [pltpu.VMEM((CH,), jnp.int32) for _ in range(2 * KA)],
        [pltpu.VMEM((CH,), jnp.int32) for _ in range(2 * KA)],
        [pltpu.VMEM((CH, D_HID), jnp.float32) for _ in range(KA)],
        pltpu.VMEM((TAIL,), jnp.int32),
        pltpu.VMEM((TAIL,), jnp.int32),
        pltpu.VMEM((TAIL, D_HID), jnp.float32),
        pltpu.VMEM((STRIPE, D_HID), jnp.float32),
        pltpu.VMEM_SHARED((NP, D_HID), jnp.float32),
        pltpu.VMEM((N // 16, D_HID), jnp.float32),
        pltpu.VMEM_SHARED((N, D_HID), jnp.float32),
        [pltpu.SemaphoreType.DMA for _ in range(2)],
        [pltpu.SemaphoreType.DMA for _ in range(KA)],
        [pltpu.SemaphoreType.DMA for _ in range(KA)],
    ],
    compiler_params=_SC_PARAMS,
)
def _sc_aggregate(tab_hbm, ei_hbm, out_hbm, srcb, dstb, rows, srct, dstt,
                  rowst, wb_v, acc, tstage, tab_sp, si, sg, ss):
    c = lax.axis_index("c")
    s = lax.axis_index("s")
    base = s * STRIPE
    _zero_fill(wb_v, STRIPE)
    pltpu.sync_copy(wb_v, acc.at[pl.ds(base, STRIPE)])
    tbase = s * (N // 16)
    pltpu.sync_copy(tab_hbm.at[pl.ds(tbase, N // 16)], tstage)
    pltpu.sync_copy(tstage, tab_sp.at[pl.ds(tbase, N // 16)])
    plsc.subcore_barrier()

    ebase = (c * 16 + s) * EPW

    def idx_at(row, g, b):
        return ei_hbm.at[row, pl.ds(ebase + (g * KA + b) * CH, CH)]

    for b in range(KA):
        pltpu.async_copy(idx_at(0, 0, b), srcb[b], si[0])
        pltpu.async_copy(idx_at(1, 0, b), dstb[b], si[0])

    def phase(g, g2, p, first, prefetch_on):
        for b in range(KA):
            pltpu.make_async_copy(idx_at(0, g, b), srcb[p * KA + b], si[p]).wait()
            pltpu.make_async_copy(idx_at(1, g, b), dstb[p * KA + b], si[p]).wait()

        def wait_scatters():
            for b in range(KA):
                pltpu.make_async_copy(rows[b], acc.at[dstb[p * KA + b]],
                                      ss[b]).wait()

        if first:
            pl.when(g2 > 0)(wait_scatters)
        else:
            wait_scatters()

        for b in range(KA):
            pltpu.async_copy(tab_sp.at[srcb[p * KA + b]], rows[b], sg[b])

        if prefetch_on:
            for b in range(KA):
                pltpu.async_copy(idx_at(0, g + 1, b), srcb[(1 - p) * KA + b],
                                 si[1 - p])
                pltpu.async_copy(idx_at(1, g + 1, b), dstb[(1 - p) * KA + b],
                                 si[1 - p])

        for b in range(KA):
            pltpu.make_async_copy(tab_sp.at[srcb[p * KA + b]], rows[b],
                                  sg[b]).wait()
            pltpu.async_copy(rows[b], acc.at[dstb[p * KA + b]], ss[b], add=True)

    def step(g2, _):
        phase(2 * g2, g2, 0, True, True)
        phase(2 * g2 + 1, g2, 1, False, True)
        return 0

    lax.fori_loop(0, GA // 2, step, 0)
    phase(GA - 1, GA // 2, 0, False, False)
    for b in range(KA):
        pltpu.make_async_copy(rows[b], acc.at[dstb[b]], ss[b]).wait()

    pltpu.sync_copy(ei_hbm.at[0, pl.ds(ebase + FULL * CH, TAIL)], srct)
    pltpu.sync_copy(ei_hbm.at[1, pl.ds(ebase + FULL * CH, TAIL)], dstt)
    pltpu.sync_copy(tab_sp.at[srct], rowst)
    pltpu.sync_copy(rowst, acc.at[dstt], add=True)

    plsc.subcore_barrier()
    pltpu.sync_copy(acc.at[pl.ds(base, STRIPE)], wb_v)
    pltpu.sync_copy(wb_v, out_hbm.at[c].at[pl.ds(base, STRIPE)])


NPK = N // 8
NPP = NP // 8


def _group_iota(shape, d0, d1):
    a = lax.broadcasted_iota(jnp.int32, shape, d0) // 16
    b = lax.broadcasted_iota(jnp.int32, shape, d1)
    return (a == b).astype(jnp.float32)


def _tc_dense1_body(x_ref, w1_ref, degp_ref, xs_ref, dinv_ref):
    w1 = w1_ref[...]
    w1t = jnp.tile(w1, (8, 8))
    r = lax.broadcasted_iota(jnp.int32, (8 * D_IN, 128), 0) // D_IN
    c = lax.broadcasted_iota(jnp.int32, (8 * D_IN, 128), 1) // D_HID
    w1bd = w1t * (r == c).astype(jnp.float32)
    xw_p = jnp.dot(x_ref[...], w1bd, preferred_element_type=jnp.float32)
    deg = degp_ref[0, :NPK] + degp_ref[1, :NPK] + 1.0
    dinv = lax.rsqrt(deg)
    xs_ref[...] = xw_p * dinv
    dinv_ref[...] = dinv


def _tc_dense2_body(ap_ref, xs_ref, dinv_ref, b1_ref, w2_ref, b2_ref, g_ref,
                    q_ref):
    a = ap_ref[0, :NPK] + ap_ref[1, :NPK]
    dinv = dinv_ref[...]
    h = jnp.maximum(dinv * a + dinv * xs_ref[...] + b1_ref[...], 0.0)
    w2 = w2_ref[...]
    wd = w2[:, 0] - w2[:, 1]
    wd128 = jnp.concatenate([wd] * 8)
    hw = h * wd128
    t8 = jnp.dot(hw, _group_iota((128, 8), 0, 1),
                 preferred_element_type=jnp.float32)
    t = jnp.dot(t8, _group_iota((8, 128), 1, 0),
                preferred_element_type=jnp.float32)
    g_ref[...] = t * dinv
    q_ref[...] = t * dinv * dinv + (b2_ref[0, 0] - b2_ref[0, 1])


def _tc_dense3_body(ap_ref, dinv_ref, q_ref, y_ref):
    a = ap_ref[0, :NPK] + ap_ref[1, :NPK]
    z = dinv_ref[...] * a + q_ref[...]
    sp_pos = jnp.maximum(z, 0.0) + jnp.log1p(jnp.exp(-jnp.abs(z)))
    sp_neg = sp_pos - z
    pick = _group_iota((128, 8), 0, 1) * (
        (lax.broadcasted_iota(jnp.int32, (128, 8), 0) % 16 == 0)
        .astype(jnp.float32))
    y0 = jnp.dot(-sp_neg, pick, preferred_element_type=jnp.float32)
    y1 = jnp.dot(-sp_pos, pick, preferred_element_type=jnp.float32)
    i0 = lax.broadcasted_iota(jnp.int32, (8, 16), 0)
    i1 = lax.broadcasted_iota(jnp.int32, (8, 16), 1)
    p0 = (i1 == 2 * i0).astype(jnp.float32)
    p1 = (i1 == 2 * i0 + 1).astype(jnp.float32)
    y_ref[...] = (jnp.dot(y0, p0, preferred_element_type=jnp.float32)
                  + jnp.dot(y1, p1, preferred_element_type=jnp.float32))


def kernel(x, edge_index, W1, b1, W2, b2):
    degp = _sc_degree(edge_index).reshape(2, NPP, 128)

    xs, dinv = pl.pallas_call(
        _tc_dense1_body,
        out_shape=(
            jax.ShapeDtypeStruct((NPK, 128), jnp.float32),
            jax.ShapeDtypeStruct((NPK, 128), jnp.float32),
        ),
    )(x.reshape(NPK, 8 * D_IN), W1, degp)

    a1 = _sc_aggregate(xs.reshape(N, D_HID), edge_index).reshape(2, NPP, 128)

    b1_128 = jnp.tile(b1.reshape(1, D_HID), (1, 8))
    g, q = pl.pallas_call(
        _tc_dense2_body,
        out_shape=(
            jax.ShapeDtypeStruct((NPK, 128), jnp.float32),
            jax.ShapeDtypeStruct((NPK, 128), jnp.float32),
        ),
    )(a1, xs, dinv, b1_128, W2, b2.reshape(1, 2))

    a2 = _sc_aggregate(g.reshape(N, D_HID), edge_index).reshape(2, NPP, 128)

    yp = pl.pallas_call(
        _tc_dense3_body,
        out_shape=jax.ShapeDtypeStruct((NPK, 16), jnp.float32),
    )(a2, dinv, q)

    return yp.reshape(N, 2)


# speedup vs baseline: 2.1426x; 1.0501x over previous
import functools

import jax
import jax.numpy as jnp
from jax import lax
from jax.experimental import pallas as pl
from jax.experimental.pallas import tpu as pltpu
from jax.experimental.pallas import tpu_sc as plsc

N = 10000
D_IN = 128
D_HID = 16
E = 320000

NW = 32
EPW = E // NW
CH = 128
K = 3
KA = 6
FULL = EPW // CH
G = FULL // K
GA = FULL // KA
TAIL = EPW - FULL * CH
NP = 10112
STRIPE = NP // 16
NP1 = 10240
STRIPE1 = NP1 // 16

_SC_MESH = plsc.VectorSubcoreMesh(core_axis_name="c", subcore_axis_name="s")
_SC_PARAMS = pltpu.CompilerParams(use_tc_tiling_on_sc=False)


def _zero_fill(ref, nrows):
    def body(i, _):
        ref[i] = jnp.zeros((D_HID,), jnp.float32)
        return 0

    lax.fori_loop(0, nrows, body, 0)


@functools.partial(
    pl.kernel,
    out_type=jax.ShapeDtypeStruct((2, NP, D_HID), jnp.float32),
    mesh=_SC_MESH,
    scratch_types=[
        [pltpu.VMEM((CH,), jnp.int32) for _ in range(2 * KA)],
        pltpu.VMEM((CH, D_HID), jnp.float32),
        pltpu.VMEM((STRIPE, D_HID), jnp.float32),
        pltpu.VMEM_SHARED((NP, D_HID), jnp.float32),
        [pltpu.SemaphoreType.DMA for _ in range(2)],
        [pltpu.SemaphoreType.DMA for _ in range(KA)],
    ],
    compiler_params=_SC_PARAMS,
)
def _sc_degree(ei_hbm, out_hbm, dstb, ones_v, wb_v, acc, si, ss):
    c = lax.axis_index("c")
    s = lax.axis_index("s")
    base = s * STRIPE
    _zero_fill(wb_v, STRIPE)
    pltpu.sync_copy(wb_v, acc.at[pl.ds(base, STRIPE)])

    def fill_ones(i, _):
        ones_v[i] = jnp.ones((D_HID,), jnp.float32)
        return 0

    lax.fori_loop(0, CH, fill_ones, 0)
    plsc.subcore_barrier()

    ebase = (c * 16 + s) * EPW

    def idx_src(g, b):
        return ei_hbm.at[1, pl.ds(ebase + (g * KA + b) * CH, CH)]

    for b in range(KA):
        pltpu.async_copy(idx_src(0, b), dstb[b], si[0])

    def phase(g, g2, p, first, prefetch_on):
        for b in range(KA):
            pltpu.make_async_copy(idx_src(g, b), dstb[p * KA + b], si[p]).wait()

        def wait_scatters():
            for b in range(KA):
                pltpu.make_async_copy(ones_v, acc.at[dstb[p * KA + b]],
                                      ss[b]).wait()

        if first:
            pl.when(g2 > 0)(wait_scatters)
        else:
            wait_scatters()

        for b in range(KA):
            pltpu.async_copy(ones_v, acc.at[dstb[p * KA + b]], ss[b], add=True)

        if prefetch_on:
            for b in range(KA):
                pltpu.async_copy(idx_src(g + 1, b), dstb[(1 - p) * KA + b],
                                 si[1 - p])

    def step(g2, _):
        phase(2 * g2, g2, 0, True, True)
        phase(2 * g2 + 1, g2, 1, False, True)
        return 0

    lax.fori_loop(0, GA // 2, step, 0)
    phase(GA - 1, GA // 2, 0, False, False)
    for b in range(KA):
        pltpu.make_async_copy(ones_v, acc.at[dstb[b]], ss[b]).wait()

    pltpu.sync_copy(ei_hbm.at[1, pl.ds(ebase + FULL * CH, TAIL)],
                    dstb[0].at[pl.ds(0, TAIL)])
    pltpu.sync_copy(ones_v.at[pl.ds(0, TAIL)],
                    acc.at[dstb[0].at[pl.ds(0, TAIL)]], add=True)

    plsc.subcore_barrier()
    pltpu.sync_copy(acc.at[pl.ds(base, STRIPE)], wb_v)
    pltpu.sync_copy(wb_v, out_hbm.at[c].at[pl.ds(base, STRIPE)])


@functools.partial(
    pl.kernel,
    out_type=jax.ShapeDtypeStruct((2, NP, D_HID), jnp.float32),
    mesh=_SC_MESH,
    scratch_types=[
        [pltpu.VMEM((CH,), jnp.int32) for _ in range(2 * KA)],
        [pltpu.VMEM((CH,), jnp.int32) for _ in range(2 * KA)],
        [pltpu.VMEM((CH, D_HID), jnp.float32) for _ in range(KA)],
        pltpu.VMEM((TAIL,), jnp.int32),
        pltpu.VMEM((TAIL,), jnp.int32),
        pltpu.VMEM((TAIL, D_HID), jnp.float32),
        pltpu.VMEM((STRIPE, D_HID), jnp.float32),
        pltpu.VMEM_SHARED((NP, D_HID), jnp.float32),
        pltpu.VMEM((N // 16, D_HID), jnp.float32),
        pltpu.VMEM_SHARED((N, D_HID), jnp.float32),
        [pltpu.SemaphoreType.DMA for _ in range(2)],
        [pltpu.SemaphoreType.DMA for _ in range(KA)],
        [pltpu.SemaphoreType.DMA for _ in range(KA)],
    ],
    compiler_params=_SC_PARAMS,
)
def _sc_aggregate(tab_hbm, ei_hbm, out_hbm, srcb, dstb, rows, srct, dstt,
                  rowst, wb_v, acc, tstage, tab_sp, si, sg, ss):
    c = lax.axis_index("c")
    s = lax.axis_index("s")
    base = s * STRIPE
    _zero_fill(wb_v, STRIPE)
    pltpu.sync_copy(wb_v, acc.at[pl.ds(base, STRIPE)])
    tbase = s * (N // 16)
    pltpu.sync_copy(tab_hbm.at[pl.ds(tbase, N // 16)], tstage)
    pltpu.sync_copy(tstage, tab_sp.at[pl.ds(tbase, N // 16)])
    plsc.subcore_barrier()

    ebase = (c * 16 + s) * EPW

    def idx_at(row, g, b):
        return ei_hbm.at[row, pl.ds(ebase + (g * KA + b) * CH, CH)]

    for b in range(KA):
        pltpu.async_copy(idx_at(0, 0, b), srcb[b], si[0])
        pltpu.async_copy(idx_at(1, 0, b), dstb[b], si[0])

    def phase(g, g2, p, first, prefetch_on):
        for b in range(KA):
            pltpu.make_async_copy(idx_at(0, g, b), srcb[p * KA + b], si[p]).wait()
            pltpu.make_async_copy(idx_at(1, g, b), dstb[p * KA + b], si[p]).wait()

        def wait_scatters():
            for b in range(KA):
                pltpu.make_async_copy(rows[b], acc.at[dstb[p * KA + b]],
                                      ss[b]).wait()

        if first:
            pl.when(g2 > 0)(wait_scatters)
        else:
            wait_scatters()

        for b in range(KA):
            pltpu.async_copy(tab_sp.at[srcb[p * KA + b]], rows[b], sg[b])

        if prefetch_on:
            for b in range(KA):
                pltpu.async_copy(idx_at(0, g + 1, b), srcb[(1 - p) * KA + b],
                                 si[1 - p])
                pltpu.async_copy(idx_at(1, g + 1, b), dstb[(1 - p) * KA + b],
                                 si[1 - p])

        for b in range(KA):
            pltpu.make_async_copy(tab_sp.at[srcb[p * KA + b]], rows[b],
                                  sg[b]).wait()
            pltpu.async_copy(rows[b], acc.at[dstb[p * KA + b]], ss[b], add=True)

    def step(g2, _):
        phase(2 * g2, g2, 0, True, True)
        phase(2 * g2 + 1, g2, 1, False, True)
        return 0

    lax.fori_loop(0, GA // 2, step, 0)
    phase(GA - 1, GA // 2, 0, False, False)
    for b in range(KA):
        pltpu.make_async_copy(rows[b], acc.at[dstb[b]], ss[b]).wait()

    pltpu.sync_copy(ei_hbm.at[0, pl.ds(ebase + FULL * CH, TAIL)], srct)
    pltpu.sync_copy(ei_hbm.at[1, pl.ds(ebase + FULL * CH, TAIL)], dstt)
    pltpu.sync_copy(tab_sp.at[srct], rowst)
    pltpu.sync_copy(rowst, acc.at[dstt], add=True)

    plsc.subcore_barrier()
    pltpu.sync_copy(acc.at[pl.ds(base, STRIPE)], wb_v)
    pltpu.sync_copy(wb_v, out_hbm.at[c].at[pl.ds(base, STRIPE)])


NPK = N // 8
NPP = NP // 8


def _group_iota(shape, d0, d1):
    a = lax.broadcasted_iota(jnp.int32, shape, d0) // 16
    b = lax.broadcasted_iota(jnp.int32, shape, d1)
    return (a == b).astype(jnp.float32)


def _tc_dense1_body(x_ref, w1_ref, degp_ref, xs_ref, dinv_ref):
    w1 = w1_ref[...]
    w1t = jnp.tile(w1, (8, 8))
    r = lax.broadcasted_iota(jnp.int32, (8 * D_IN, 128), 0) // D_IN
    c = lax.broadcasted_iota(jnp.int32, (8 * D_IN, 128), 1) // D_HID
    w1bd = w1t * (r == c).astype(jnp.float32)
    xw_p = jnp.dot(x_ref[...], w1bd, preferred_element_type=jnp.float32)
    deg = degp_ref[0, :NPK] + degp_ref[1, :NPK] + 1.0
    dinv = lax.rsqrt(deg)
    xs_ref[...] = xw_p * dinv
    dinv_ref[...] = dinv


def _tc_dense2_body(ap_ref, xs_ref, dinv_ref, b1_ref, w2_ref, b2_ref, g_ref,
                    q_ref):
    a = ap_ref[0, :NPK] + ap_ref[1, :NPK]
    dinv = dinv_ref[...]
    h = jnp.maximum(dinv * a + dinv * xs_ref[...] + b1_ref[...], 0.0)
    w2 = w2_ref[...]
    wd = w2[:, 0] - w2[:, 1]
    wd128 = jnp.concatenate([wd] * 8)
    hw = h * wd128
    t8 = jnp.dot(hw, _group_iota((128, 8), 0, 1),
                 preferred_element_type=jnp.float32)
    t = jnp.dot(t8, _group_iota((8, 128), 1, 0),
                preferred_element_type=jnp.float32)
    g_ref[...] = t * dinv
    q_ref[...] = t * dinv * dinv + (b2_ref[0, 0] - b2_ref[0, 1])


def _tc_dense3_body(ap_ref, dinv_ref, q_ref, y_ref):
    a = ap_ref[0, :NPK] + ap_ref[1, :NPK]
    z = dinv_ref[...] * a + q_ref[...]
    sp_pos = jnp.maximum(z, 0.0) + jnp.log1p(jnp.exp(-jnp.abs(z)))
    sp_neg = sp_pos - z
    pick = _group_iota((128, 8), 0, 1) * (
        (lax.broadcasted_iota(jnp.int32, (128, 8), 0) % 16 == 0)
        .astype(jnp.float32))
    y0 = jnp.dot(-sp_neg, pick, preferred_element_type=jnp.float32)
    y1 = jnp.dot(-sp_pos, pick, preferred_element_type=jnp.float32)
    i0 = lax.broadcasted_iota(jnp.int32, (8, 16), 0)
    i1 = lax.broadcasted_iota(jnp.int32, (8, 16), 1)
    p0 = (i1 == 2 * i0).astype(jnp.float32)
    p1 = (i1 == 2 * i0 + 1).astype(jnp.float32)
    y_ref[...] = (jnp.dot(y0, p0, preferred_element_type=jnp.float32)
                  + jnp.dot(y1, p1, preferred_element_type=jnp.float32))


def kernel(x, edge_index, W1, b1, W2, b2):
    degp = _sc_degree(edge_index).reshape(2, NPP, 128)

    xs, dinv = pl.pallas_call(
        _tc_dense1_body,
        out_shape=(
            jax.ShapeDtypeStruct((NPK, 128), jnp.float32),
            jax.ShapeDtypeStruct((NPK, 128), jnp.float32),
        ),
    )(x.reshape(NPK, 8 * D_IN), W1, degp)

    a1 = _sc_aggregate(xs.reshape(N, D_HID), edge_index).reshape(2, NPP, 128)

    b1_128 = jnp.tile(b1.reshape(1, D_HID), (1, 8))
    g, q = pl.pallas_call(
        _tc_dense2_body,
        out_shape=(
            jax.ShapeDtypeStruct((NPK, 128), jnp.float32),
            jax.ShapeDtypeStruct((NPK, 128), jnp.float32),
        ),
    )(a1, xs, dinv, b1_128, W2, b2.reshape(1, 2))

    a2 = _sc_aggregate(g.reshape(N, D_HID), edge_index).reshape(2, NPP, 128)

    yp = pl.pallas_call(
        _tc_dense3_body,
        out_shape=jax.ShapeDtypeStruct((NPK, 16), jnp.float32),
    )(a2, dinv, q)

    return yp.reshape(N, 2)
